# baseline (speedup 1.0000x reference)
import functools

import jax
import jax.numpy as jnp
from jax import lax
from jax.experimental import pallas as pl
from jax.experimental.pallas import tpu as pltpu
from jax.experimental.pallas import tpu_sc as plsc


def _make_sc_aggregate(N, E, D):
  info = plsc.get_sparse_core_info()
  NC, NS = info.num_cores, info.num_subcores
  NW = NC * NS
  per_tile = E // NW
  C = next(c for c in range(50, 0, -1)
           if per_tile % c == 0 and (per_tile // c) % 4 == 0
           and (per_tile // c) * 128 + 2 * c * D <= 45000)
  n_chunks = per_tile // C
  n_half = n_chunks // 2
  R8 = (N // NS) & ~7
  rem = N - NS * R8

  mesh = plsc.VectorSubcoreMesh(core_axis_name="c", subcore_axis_name="s")

  @functools.partial(
      pl.kernel,
      out_type=jax.ShapeDtypeStruct((NC, N, D), jnp.float32),
      mesh=mesh,
      scratch_types=[
          pltpu.VMEM((n_half, C), jnp.int32),
          pltpu.VMEM((n_half, C), jnp.int32),
          pltpu.VMEM((C, D), jnp.float32),
          pltpu.VMEM((C, D), jnp.float32),
          pltpu.VMEM_SHARED((N, D), jnp.float32),
          pltpu.SemaphoreType.DMA,
          pltpu.SemaphoreType.DMA,
      ],
  )
  def agg(x_hbm, e_hbm, out_hbm, src_half, dst_half, rows_a,
          rows_b, acc, sem_a, sem_b):
    cid = lax.axis_index("c")
    sid = lax.axis_index("s")
    wid = cid * NS + sid

    def gather(j, buf, sem):
      return pltpu.async_copy(x_hbm.at[src_half.at[j]], buf, sem)

    pltpu.sync_copy(e_hbm.at[0, wid, 0], src_half)
    pltpu.sync_copy(e_hbm.at[1, wid, 0], dst_half)
    gather(0, rows_a, sem_a)

    zrows = (C // 8) * 8
    zero = jnp.zeros((16,), jnp.float32)

    @pl.loop(0, zrows)
    def _(r):
      for cc in range(D // 16):
        rows_b[r, pl.ds(cc * 16, 16)] = zero

    def zero_fill(base_row, nrows):
      full, last = nrows // zrows, nrows % zrows
      for k in range(full):
        pltpu.sync_copy(rows_b.at[pl.ds(0, zrows)],
                        acc.at[pl.ds(base_row + k * zrows, zrows)])
      if last:
        pltpu.sync_copy(rows_b.at[pl.ds(0, last)],
                        acc.at[pl.ds(base_row + full * zrows, last)])

    zero_fill(sid * R8, R8)
    if rem:
      @pl.when(sid == 0)
      def _():
        zero_fill(NS * R8, rem)
    stripe = pl.ds(sid * R8, R8)
    plsc.subcore_barrier()

    for h in range(2):
      if h:
        pltpu.sync_copy(e_hbm.at[0, wid, h], src_half)
        pltpu.sync_copy(e_hbm.at[1, wid, h], dst_half)
        gather(0, rows_a, sem_a)

      @pl.loop(0, n_half, step=2)
      def _(i):
        cp_b = gather(i + 1, rows_b, sem_b)
        pltpu.make_async_copy(x_hbm.at[src_half.at[i]], rows_a, sem_a).wait()
        pltpu.sync_copy(rows_a, acc.at[dst_half.at[i]], add=True)

        @pl.when(i + 2 < n_half)
        def _():
          gather(i + 2, rows_a, sem_a)

        cp_b.wait()
        pltpu.sync_copy(rows_b, acc.at[dst_half.at[i + 1]], add=True)

    plsc.subcore_barrier()
    pltpu.sync_copy(acc.at[stripe], out_hbm.at[cid, stripe])
    if rem:
      @pl.when(sid == 0)
      def _():
        tail = pl.ds(NS * R8, rem)
        pltpu.sync_copy(acc.at[tail], out_hbm.at[cid, tail])

  return agg, NW, n_half, C


def _mlp(x, accs, W1, b1, W2, b2, scale):
  N, D = x.shape
  H = W2.shape[1]
  BN = 5000
  grid = N // BN

  def body(x_ref, a_ref, w1_ref, b1_ref, w2_ref, b2_ref, s_ref, o_ref):
    xb = x_ref[...]
    t = a_ref[0] + a_ref[1] + xb
    h = jnp.dot(t, w1_ref[...], preferred_element_type=jnp.float32)
    h = jnp.maximum(h + b1_ref[...], 0.0)
    h = jnp.dot(h, w2_ref[...], preferred_element_type=jnp.float32)
    h = jnp.maximum(h + b2_ref[...], 0.0)
    o_ref[...] = h + s_ref[0] * xb

  return pl.pallas_call(
      body,
      grid=(grid,),
      in_specs=[
          pl.BlockSpec((BN, D), lambda i: (i, 0)),
          pl.BlockSpec((2, BN, D), lambda i: (0, i, 0)),
          pl.BlockSpec((D, H), lambda i: (0, 0)),
          pl.BlockSpec((1, H), lambda i: (0, 0)),
          pl.BlockSpec((H, H), lambda i: (0, 0)),
          pl.BlockSpec((1, H), lambda i: (0, 0)),
          pl.BlockSpec(memory_space=pltpu.SMEM),
      ],
      out_specs=pl.BlockSpec((BN, H), lambda i: (i, 0)),
      out_shape=jax.ShapeDtypeStruct((N, H), jnp.float32),
  )(x, accs, W1, b1.reshape(1, H), W2, b2.reshape(1, H), scale)


def kernel(x, edge_index, W1, b1, W2, b2, epsilon):
  N, D = x.shape
  E = edge_index.shape[1]
  agg, nw, n_half, c = _make_sc_aggregate(N, E, D)
  edges = edge_index.reshape(2, nw, 2, n_half, c)
  accs = agg(x, edges)
  scale = jnp.reshape(1.0 + epsilon, (1,)).astype(jnp.float32)
  return _mlp(x, accs, W1, b1, W2, b2, scale)


# speedup vs baseline: 1.3639x; 1.3639x over previous
import functools

import jax
import jax.numpy as jnp
from jax import lax
from jax.experimental import pallas as pl
from jax.experimental.pallas import tpu as pltpu
from jax.experimental.pallas import tpu_sc as plsc


def _make_sc_aggregate(N, E, D):
  info = plsc.get_sparse_core_info()
  NC, NS = info.num_cores, info.num_subcores
  NW = NC * NS
  per_tile = E // NW
  C = next(c for c in range(128, 0, -1)
           if per_tile % (4 * c) == 0
           and (per_tile // (4 * c)) * 128 * 2 + 3 * c * D <= 45000)
  n_q = per_tile // (4 * C)
  R8 = (N // NS) & ~7
  rem = N - NS * R8

  mesh = plsc.VectorSubcoreMesh(core_axis_name="c", subcore_axis_name="s")

  @functools.partial(
      pl.kernel,
      out_type=jax.ShapeDtypeStruct((NC, N, D), jnp.float32),
      mesh=mesh,
      scratch_types=[
          pltpu.VMEM((n_q, C), jnp.int32),
          pltpu.VMEM((n_q, C), jnp.int32),
          pltpu.VMEM((C, D), jnp.float32),
          pltpu.VMEM((C, D), jnp.float32),
          pltpu.VMEM((C, D), jnp.float32),
          pltpu.VMEM_SHARED((N, D), jnp.float32),
          pltpu.SemaphoreType.DMA,
          pltpu.SemaphoreType.DMA,
          pltpu.SemaphoreType.DMA,
      ],
  )
  def agg(x_hbm, e_hbm, out_hbm, src_q, dst_q, rows_0, rows_1, rows_2,
          acc, sem_0, sem_1, sem_2):
    cid = lax.axis_index("c")
    sid = lax.axis_index("s")
    wid = cid * NS + sid
    bufs = ((rows_0, sem_0), (rows_1, sem_1), (rows_2, sem_2))

    def gather(j, b):
      return pltpu.async_copy(x_hbm.at[src_q.at[j]], bufs[b][0], bufs[b][1])

    def wait_gather(j, b):
      pltpu.make_async_copy(x_hbm.at[src_q.at[j]], bufs[b][0],
                            bufs[b][1]).wait()

    def scatter(j, b):
      pltpu.sync_copy(bufs[b][0], acc.at[dst_q.at[j]], add=True)

    pltpu.sync_copy(e_hbm.at[0, wid, 0], src_q)
    pltpu.sync_copy(e_hbm.at[1, wid, 0], dst_q)
    gather(0, 0)
    if n_q > 1:
      gather(1, 1)

    zrows = (C // 8) * 8
    zero = jnp.zeros((16,), jnp.float32)

    @pl.loop(0, zrows)
    def _(r):
      for cc in range(D // 16):
        rows_2[r, pl.ds(cc * 16, 16)] = zero

    def zero_fill(base_row, nrows):
      full, last = nrows // zrows, nrows % zrows
      for k in range(full):
        pltpu.sync_copy(rows_2.at[pl.ds(0, zrows)],
                        acc.at[pl.ds(base_row + k * zrows, zrows)])
      if last:
        pltpu.sync_copy(rows_2.at[pl.ds(0, last)],
                        acc.at[pl.ds(base_row + full * zrows, last)])

    zero_fill(sid * R8, R8)
    if rem:
      @pl.when(sid == 0)
      def _():
        zero_fill(NS * R8, rem)
    stripe = pl.ds(sid * R8, R8)
    plsc.subcore_barrier()

    for q in range(4):
      if q:
        pltpu.sync_copy(e_hbm.at[0, wid, q], src_q)
        pltpu.sync_copy(e_hbm.at[1, wid, q], dst_q)
        gather(0, 0)
        if n_q > 1:
          gather(1, 1)

      @pl.loop(0, n_q, step=3)
      def _(j):
        wait_gather(j, 0)

        @pl.when(j + 2 < n_q)
        def _():
          gather(j + 2, 2)

        scatter(j, 0)

        @pl.when(j + 1 < n_q)
        def _():
          wait_gather(j + 1, 1)

          @pl.when(j + 3 < n_q)
          def _():
            gather(j + 3, 0)

          scatter(j + 1, 1)

        @pl.when(j + 2 < n_q)
        def _():
          wait_gather(j + 2, 2)

          @pl.when(j + 4 < n_q)
          def _():
            gather(j + 4, 1)

          scatter(j + 2, 2)

    plsc.subcore_barrier()
    pltpu.sync_copy(acc.at[stripe], out_hbm.at[cid, stripe])
    if rem:
      @pl.when(sid == 0)
      def _():
        tail = pl.ds(NS * R8, rem)
        pltpu.sync_copy(acc.at[tail], out_hbm.at[cid, tail])

  return agg, NW, n_q, C


def _mlp(x, accs, W1, b1, W2, b2, scale):
  N, D = x.shape
  H = W2.shape[1]
  BN = 5000
  grid = N // BN

  def body(x_ref, a_ref, w1_ref, b1_ref, w2_ref, b2_ref, s_ref, o_ref):
    xb = x_ref[...]
    t = a_ref[0] + a_ref[1] + xb
    h = jnp.dot(t, w1_ref[...], preferred_element_type=jnp.float32)
    h = jnp.maximum(h + b1_ref[...], 0.0)
    h = jnp.dot(h, w2_ref[...], preferred_element_type=jnp.float32)
    h = jnp.maximum(h + b2_ref[...], 0.0)
    o_ref[...] = h + s_ref[0] * xb

  return pl.pallas_call(
      body,
      grid=(grid,),
      in_specs=[
          pl.BlockSpec((BN, D), lambda i: (i, 0)),
          pl.BlockSpec((2, BN, D), lambda i: (0, i, 0)),
          pl.BlockSpec((D, H), lambda i: (0, 0)),
          pl.BlockSpec((1, H), lambda i: (0, 0)),
          pl.BlockSpec((H, H), lambda i: (0, 0)),
          pl.BlockSpec((1, H), lambda i: (0, 0)),
          pl.BlockSpec(memory_space=pltpu.SMEM),
      ],
      out_specs=pl.BlockSpec((BN, H), lambda i: (i, 0)),
      out_shape=jax.ShapeDtypeStruct((N, H), jnp.float32),
  )(x, accs, W1, b1.reshape(1, H), W2, b2.reshape(1, H), scale)


def kernel(x, edge_index, W1, b1, W2, b2, epsilon):
  N, D = x.shape
  E = edge_index.shape[1]
  agg, nw, n_q, c = _make_sc_aggregate(N, E, D)
  edges = edge_index.reshape(2, nw, 4, n_q, c)
  accs = agg(x, edges)
  scale = jnp.reshape(1.0 + epsilon, (1,)).astype(jnp.float32)
  return _mlp(x, accs, W1, b1, W2, b2, scale)


# speedup vs baseline: 1.3661x; 1.0016x over previous
import functools

import jax
import jax.numpy as jnp
from jax import lax
from jax.experimental import pallas as pl
from jax.experimental.pallas import tpu as pltpu
from jax.experimental.pallas import tpu_sc as plsc


def _make_sc_aggregate(N, E, D):
  info = plsc.get_sparse_core_info()
  NC, NS = info.num_cores, info.num_subcores
  NW = NC * NS
  per_tile = E // NW
  NSTAGE = 4
  C = next(c for c in range(128, 0, -1)
           if per_tile % (NSTAGE * c) == 0
           and (per_tile // (NSTAGE * c)) * 128 * 2 + 3 * c * D <= 45000)
  n_q = per_tile // (NSTAGE * C)
  R8 = (N // NS) & ~7
  rem = N - NS * R8

  mesh = plsc.VectorSubcoreMesh(core_axis_name="c", subcore_axis_name="s")

  @functools.partial(
      pl.kernel,
      out_type=jax.ShapeDtypeStruct((NC, N, D), jnp.float32),
      mesh=mesh,
      scratch_types=[
          pltpu.VMEM((n_q, C), jnp.int32),
          pltpu.VMEM((n_q, C), jnp.int32),
          pltpu.VMEM((C, D), jnp.float32),
          pltpu.VMEM((C, D), jnp.float32),
          pltpu.VMEM((C, D), jnp.float32),
          pltpu.VMEM_SHARED((N, D), jnp.float32),
          pltpu.SemaphoreType.DMA,
          pltpu.SemaphoreType.DMA,
          pltpu.SemaphoreType.DMA,
      ],
  )
  def agg(x_hbm, e_hbm, out_hbm, src_q, dst_q, rows_0, rows_1, rows_2,
          acc, sem_0, sem_1, sem_2):
    cid = lax.axis_index("c")
    sid = lax.axis_index("s")
    wid = cid * NS + sid
    bufs = ((rows_0, sem_0), (rows_1, sem_1), (rows_2, sem_2))

    def gather(j, b):
      return pltpu.async_copy(x_hbm.at[src_q.at[j]], bufs[b][0], bufs[b][1])

    def wait_gather(j, b):
      pltpu.make_async_copy(x_hbm.at[src_q.at[j]], bufs[b][0],
                            bufs[b][1]).wait()

    def scatter(j, b):
      pltpu.sync_copy(bufs[b][0], acc.at[dst_q.at[j]], add=True)

    pltpu.sync_copy(e_hbm.at[0, wid, 0], src_q)
    pltpu.sync_copy(e_hbm.at[1, wid, 0], dst_q)
    gather(0, 0)
    if n_q > 1:
      gather(1, 1)

    zrows = (C // 8) * 8
    zero = jnp.zeros((16,), jnp.float32)

    @pl.loop(0, zrows)
    def _(r):
      for cc in range(D // 16):
        rows_2[r, pl.ds(cc * 16, 16)] = zero

    def zero_fill(base_row, nrows):
      full, last = nrows // zrows, nrows % zrows
      for k in range(full):
        pltpu.sync_copy(rows_2.at[pl.ds(0, zrows)],
                        acc.at[pl.ds(base_row + k * zrows, zrows)])
      if last:
        pltpu.sync_copy(rows_2.at[pl.ds(0, last)],
                        acc.at[pl.ds(base_row + full * zrows, last)])

    zero_fill(sid * R8, R8)
    if rem:
      @pl.when(sid == 0)
      def _():
        zero_fill(NS * R8, rem)
    stripe = pl.ds(sid * R8, R8)
    plsc.subcore_barrier()

    for q in range(NSTAGE):
      if q:
        pltpu.sync_copy(e_hbm.at[0, wid, q], src_q)
        pltpu.sync_copy(e_hbm.at[1, wid, q], dst_q)
        gather(0, 0)
        if n_q > 1:
          gather(1, 1)

      @pl.loop(0, n_q, step=3)
      def _(j):
        wait_gather(j, 0)

        @pl.when(j + 2 < n_q)
        def _():
          gather(j + 2, 2)

        scatter(j, 0)

        @pl.when(j + 1 < n_q)
        def _():
          wait_gather(j + 1, 1)

          @pl.when(j + 3 < n_q)
          def _():
            gather(j + 3, 0)

          scatter(j + 1, 1)

        @pl.when(j + 2 < n_q)
        def _():
          wait_gather(j + 2, 2)

          @pl.when(j + 4 < n_q)
          def _():
            gather(j + 4, 1)

          scatter(j + 2, 2)

    plsc.subcore_barrier()
    pltpu.sync_copy(acc.at[stripe], out_hbm.at[cid, stripe])
    if rem:
      @pl.when(sid == 0)
      def _():
        tail = pl.ds(NS * R8, rem)
        pltpu.sync_copy(acc.at[tail], out_hbm.at[cid, tail])

  return agg, NW, NSTAGE, n_q, C


def _mlp(x, accs, W1, b1, W2, b2, scale):
  N, D = x.shape
  H = W2.shape[1]
  BN = 5000
  grid = N // BN

  def body(x_ref, a_ref, w1_ref, b1_ref, w2_ref, b2_ref, s_ref, o_ref):
    xb = x_ref[...]
    t = a_ref[0] + a_ref[1] + xb
    h = jnp.dot(t, w1_ref[...], preferred_element_type=jnp.float32)
    h = jnp.maximum(h + b1_ref[...], 0.0)
    h = jnp.dot(h, w2_ref[...], preferred_element_type=jnp.float32)
    h = jnp.maximum(h + b2_ref[...], 0.0)
    o_ref[...] = h + s_ref[0] * xb

  return pl.pallas_call(
      body,
      grid=(grid,),
      in_specs=[
          pl.BlockSpec((BN, D), lambda i: (i, 0)),
          pl.BlockSpec((2, BN, D), lambda i: (0, i, 0)),
          pl.BlockSpec((D, H), lambda i: (0, 0)),
          pl.BlockSpec((1, H), lambda i: (0, 0)),
          pl.BlockSpec((H, H), lambda i: (0, 0)),
          pl.BlockSpec((1, H), lambda i: (0, 0)),
          pl.BlockSpec(memory_space=pltpu.SMEM),
      ],
      out_specs=pl.BlockSpec((BN, H), lambda i: (i, 0)),
      out_shape=jax.ShapeDtypeStruct((N, H), jnp.float32),
  )(x, accs, W1, b1.reshape(1, H), W2, b2.reshape(1, H), scale)


def kernel(x, edge_index, W1, b1, W2, b2, epsilon):
  N, D = x.shape
  E = edge_index.shape[1]
  agg, nw, nstage, n_q, c = _make_sc_aggregate(N, E, D)
  edges = edge_index.reshape(2, nw, nstage, n_q, c)
  accs = agg(x, edges)
  scale = jnp.reshape(1.0 + epsilon, (1,)).astype(jnp.float32)
  return _mlp(x, accs, W1, b1, W2, b2, scale)
